# Initial kernel scaffold; baseline (speedup 1.0000x reference)
import jax
import jax.numpy as jnp
from jax.experimental import pallas as pl


def kernel(x, enc_nW1, enc_nb1, enc_nW2, enc_nb2, enc_eW1, enc_eb1, enc_eW2, enc_eb2, prc_eW1, prc_eb1, prc_eW2, prc_eb2, prc_nW1, prc_nb1, prc_nW2, prc_nb2, dec_W1, dec_b1, dec_W2, dec_b2, edge_index):
    raise NotImplementedError("write your pallas kernel here")


# speedup vs baseline: 1.7605x; 1.7605x over previous
import functools

import jax
import jax.numpy as jnp
from jax import lax
from jax.experimental import pallas as pl
from jax.experimental.pallas import tpu as pltpu
from jax.experimental.pallas import tpu_sc as plsc

H = 128
NC = 2
NS = 16
NW = NC * NS
CH = 128
NCH = 40
EPW = CH * NCH
E_PAD = NW * EPW
RPT = 632
AGG_ROWS = NS * RPT


def _f32dot(a, b):
    return jnp.dot(a, b, preferred_element_type=jnp.float32)


def _encode_body(x_ref, w1_ref, b1_ref, w2_ref, b2_ref, h_ref):
    t = jnp.maximum(_f32dot(x_ref[...], w1_ref[...]) + b1_ref[...], 0.0)
    h_ref[...] = _f32dot(t, w2_ref[...]) + b2_ref[...]


def _edge_body(hs_ref, hd_ref, ws_ref, wd_ref, eb1_ref, ew2_ref, eb2_ref,
               p3_ref, pb1_ref, pw2_ref, pb2_ref, out_ref):
    pre = _f32dot(hs_ref[...], ws_ref[...]) + _f32dot(hd_ref[...], wd_ref[...])
    ea = _f32dot(jnp.maximum(pre[:, :H] + eb1_ref[...], 0.0), ew2_ref[...]) + eb2_ref[...]
    pre2 = pre[:, H:] + _f32dot(ea, p3_ref[...]) + pb1_ref[...]
    m = _f32dot(jnp.maximum(pre2, 0.0), pw2_ref[...]) + pb2_ref[...]
    out_ref[...] = ea + m


def _node_out_body(h_ref, a0_ref, a1_ref, n1h_ref, n1a_ref, nb1_ref, nw2_ref,
                   nb2_ref, dw1_ref, db1_ref, dw2_ref, db2_ref, out_ref):
    h = h_ref[...]
    agg = a0_ref[...] + a1_ref[...]
    pre = _f32dot(h, n1h_ref[...]) + _f32dot(agg, n1a_ref[...]) + nb1_ref[...]
    h2 = h + _f32dot(jnp.maximum(pre, 0.0), nw2_ref[...]) + nb2_ref[...]
    d = jnp.maximum(_f32dot(h2, dw1_ref[...]) + db1_ref[...], 0.0)
    out_ref[...] = _f32dot(d, dw2_ref[...]) + db2_ref[...]


_SC_MESH = plsc.VectorSubcoreMesh(core_axis_name="c", subcore_axis_name="s")


def _gather_body(h_tab, sidx, didx, gs_out, gd_out,
                 sidx_v, didx_v, srows, drows, sem_s, sem_d):
    c = lax.axis_index("c")
    s = lax.axis_index("s")
    wid = c * NS + s
    base = wid * EPW
    pltpu.sync_copy(sidx.at[wid], sidx_v)
    pltpu.sync_copy(didx.at[wid], didx_v)

    def body(j, carry):
        cp_s = pltpu.async_copy(h_tab.at[sidx_v.at[j]], srows, sem_s)
        cp_d = pltpu.async_copy(h_tab.at[didx_v.at[j]], drows, sem_d)
        cp_s.wait()
        pltpu.sync_copy(srows, gs_out.at[pl.ds(base + j * CH, CH)])
        cp_d.wait()
        pltpu.sync_copy(drows, gd_out.at[pl.ds(base + j * CH, CH)])
        return carry

    lax.fori_loop(0, NCH, body, 0)


def _sc_gather(h_tab, sidx, didx):
    k = pl.kernel(
        _gather_body,
        mesh=_SC_MESH,
        out_type=[
            jax.ShapeDtypeStruct((E_PAD, H), jnp.float32),
            jax.ShapeDtypeStruct((E_PAD, H), jnp.float32),
        ],
        scratch_types=[
            pltpu.VMEM((NCH, CH), jnp.int32),
            pltpu.VMEM((NCH, CH), jnp.int32),
            pltpu.VMEM((CH, H), jnp.float32),
            pltpu.VMEM((CH, H), jnp.float32),
            pltpu.SemaphoreType.DMA,
            pltpu.SemaphoreType.DMA,
        ],
    )
    return k(h_tab, sidx, didx)


def _scatter_body(eout, didx, zrows, agg_out, didx_v, rows_v, shared):
    c = lax.axis_index("c")
    s = lax.axis_index("s")
    wid = c * NS + s
    base = wid * EPW
    pltpu.sync_copy(zrows, shared.at[pl.ds(s * RPT, RPT)])
    pltpu.sync_copy(didx.at[wid], didx_v)
    plsc.subcore_barrier()

    def body(j, carry):
        pltpu.sync_copy(eout.at[pl.ds(base + j * CH, CH)], rows_v)
        pltpu.sync_copy(rows_v, shared.at[didx_v.at[j]], add=True)
        return carry

    lax.fori_loop(0, NCH, body, 0)
    plsc.subcore_barrier()
    pltpu.sync_copy(shared.at[pl.ds(s * RPT, RPT)],
                    agg_out.at[c].at[pl.ds(s * RPT, RPT)])


def _sc_scatter(eout, didx, zrows):
    k = pl.kernel(
        _scatter_body,
        mesh=_SC_MESH,
        out_type=jax.ShapeDtypeStruct((NC, AGG_ROWS, H), jnp.float32),
        scratch_types=[
            pltpu.VMEM((NCH, CH), jnp.int32),
            pltpu.VMEM((CH, H), jnp.float32),
            pltpu.VMEM_SHARED((AGG_ROWS, H), jnp.float32),
        ],
    )
    return k(eout, didx, zrows)


def kernel(x, enc_nW1, enc_nb1, enc_nW2, enc_nb2, enc_eW1, enc_eb1, enc_eW2,
           enc_eb2, prc_eW1, prc_eb1, prc_eW2, prc_eb2, prc_nW1, prc_nb1,
           prc_nW2, prc_nb2, dec_W1, dec_b1, dec_W2, dec_b2, edge_index):
    N, D_IN = x.shape
    E = edge_index.shape[1]
    OUT = dec_W2.shape[1]
    BN = 1000
    BE = 1024
    src = edge_index[0]
    dst = edge_index[1]

    r = lambda b: b.reshape(1, -1)
    full = lambda shape: pl.BlockSpec(shape, lambda i: (0, 0))

    h = pl.pallas_call(
        _encode_body,
        grid=(N // BN,),
        in_specs=[
            pl.BlockSpec((BN, D_IN), lambda i: (i, 0)),
            full((D_IN, H)), full((1, H)), full((H, H)), full((1, H)),
        ],
        out_specs=pl.BlockSpec((BN, H), lambda i: (i, 0)),
        out_shape=jax.ShapeDtypeStruct((N, H), jnp.float32),
    )(x, enc_nW1, r(enc_nb1), enc_nW2, r(enc_nb2))

    pad = E_PAD - E
    sidx = jnp.concatenate([src, jnp.zeros((pad,), jnp.int32)]).reshape(NW, NCH, CH)
    didx_g = jnp.concatenate([dst, jnp.zeros((pad,), jnp.int32)]).reshape(NW, NCH, CH)
    didx_s = jnp.concatenate(
        [dst, jnp.full((pad,), AGG_ROWS - 1, jnp.int32)]).reshape(NW, NCH, CH)

    hs, hd = _sc_gather(h, sidx, didx_g)

    Ws = jnp.concatenate([enc_eW1[:H], prc_eW1[:H]], axis=1)
    Wd = jnp.concatenate([enc_eW1[H:], prc_eW1[H:2 * H]], axis=1)
    P3 = prc_eW1[2 * H:]
    eout = pl.pallas_call(
        _edge_body,
        grid=(E_PAD // BE,),
        in_specs=[
            pl.BlockSpec((BE, H), lambda i: (i, 0)),
            pl.BlockSpec((BE, H), lambda i: (i, 0)),
            full((H, 2 * H)), full((H, 2 * H)), full((1, H)), full((H, H)),
            full((1, H)), full((H, H)), full((1, H)), full((H, H)), full((1, H)),
        ],
        out_specs=pl.BlockSpec((BE, H), lambda i: (i, 0)),
        out_shape=jax.ShapeDtypeStruct((E_PAD, H), jnp.float32),
    )(hs, hd, Ws, Wd, r(enc_eb1), enc_eW2, r(enc_eb2), P3, r(prc_eb1),
      prc_eW2, r(prc_eb2))

    zrows = jnp.zeros((RPT, H), jnp.float32)
    aggp = _sc_scatter(eout, didx_s, zrows)

    dW2p = jnp.pad(dec_W2, ((0, 0), (0, 8 - OUT)))
    db2p = jnp.pad(dec_b2, (0, 8 - OUT))
    out8 = pl.pallas_call(
        _node_out_body,
        grid=(N // BN,),
        in_specs=[
            pl.BlockSpec((BN, H), lambda i: (i, 0)),
            pl.BlockSpec((BN, H), lambda i: (i, 0)),
            pl.BlockSpec((BN, H), lambda i: (i, 0)),
            full((H, H)), full((H, H)), full((1, H)), full((H, H)),
            full((1, H)), full((H, H)), full((1, H)), full((H, 8)), full((1, 8)),
        ],
        out_specs=pl.BlockSpec((BN, 8), lambda i: (i, 0)),
        out_shape=jax.ShapeDtypeStruct((N, 8), jnp.float32),
    )(h, aggp[0, :N], aggp[1, :N], prc_nW1[:H], prc_nW1[H:], r(prc_nb1),
      prc_nW2, r(prc_nb2), dec_W1, r(dec_b1), dW2p, r(db2p))
    return out8[:, :OUT]
